# baseline (speedup 1.0000x reference)
import functools

import jax
import jax.numpy as jnp
from jax import lax
from jax.experimental import pallas as pl
from jax.experimental.pallas import tpu as pltpu
from jax.experimental.pallas import tpu_sc as plsc

B, T, D = 1, 2048, 768
H = 12
HD = D // H
E = 8
HID = int(4 * D * 2 / 3)

TT = 256
TILE = 256
NTILES = 23
NSLOT = NTILES * TILE
NEG = -1e30


def _k1_body(x_ref, g_ref, b_ref, w_ref, q_ref, k_ref, v_ref):
    x = x_ref[...]
    mu = jnp.mean(x, axis=1, keepdims=True)
    var = jnp.mean((x - mu) ** 2, axis=1, keepdims=True)
    xn = (x - mu) * lax.rsqrt(var + 1e-5) * g_ref[...] + b_ref[...]
    qkv = jnp.dot(xn.astype(jnp.bfloat16), w_ref[...],
                  preferred_element_type=jnp.float32).astype(jnp.bfloat16)
    for h in range(H):
        q_ref[h] = qkv[:, h * HD:(h + 1) * HD]
        k_ref[h] = qkv[:, D + h * HD:D + (h + 1) * HD]
        v_ref[h] = qkv[:, 2 * D + h * HD:2 * D + (h + 1) * HD]


def _qkv(x2, ln1_g, ln1_b, wqkv):
    return pl.pallas_call(
        _k1_body,
        grid=(T // TT,),
        in_specs=[
            pl.BlockSpec((TT, D), lambda i: (i, 0)),
            pl.BlockSpec((1, D), lambda i: (0, 0)),
            pl.BlockSpec((1, D), lambda i: (0, 0)),
            pl.BlockSpec((D, 3 * D), lambda i: (0, 0)),
        ],
        out_specs=[
            pl.BlockSpec((H, TT, HD), lambda i: (0, i, 0)),
            pl.BlockSpec((H, TT, HD), lambda i: (0, i, 0)),
            pl.BlockSpec((H, TT, HD), lambda i: (0, i, 0)),
        ],
        out_shape=[
            jax.ShapeDtypeStruct((H, T, HD), jnp.bfloat16),
            jax.ShapeDtypeStruct((H, T, HD), jnp.bfloat16),
            jax.ShapeDtypeStruct((H, T, HD), jnp.bfloat16),
        ],
    )(x2, ln1_g.reshape(1, D), ln1_b.reshape(1, D), wqkv)


TTA = 1024
NBQ = T // TTA
NSTEP = NBQ * (NBQ + 1) // 2
HB = 2


def _k2_body(imap_ref, jmap_ref, q_ref, k_ref, v_ref, o_ref,
             acc_ref, m_ref, l_ref):
    s_idx = pl.program_id(1)
    i = imap_ref[s_idx]
    j = jmap_ref[s_idx]

    @pl.when(j == 0)
    def _():
        acc_ref[...] = jnp.zeros_like(acc_ref)
        m_ref[...] = jnp.full_like(m_ref, NEG)
        l_ref[...] = jnp.zeros_like(l_ref)

    rowL = lax.broadcasted_iota(jnp.int32, (TTA, TTA), 0)
    colL = lax.broadcasted_iota(jnp.int32, (TTA, TTA), 1)
    diag = j == i
    for hh in range(HB):
        q = q_ref[hh] * jnp.bfloat16(1.0 / (HD ** 0.5))
        k = k_ref[hh]
        s = lax.dot_general(q, k, (((1,), (1,)), ((), ())),
                            preferred_element_type=jnp.float32)
        s = jnp.where(jnp.logical_or(jnp.logical_not(diag), colL <= rowL),
                      s, NEG)
        m_run = m_ref[hh]
        m_new = jnp.maximum(m_run, jnp.max(s, axis=1, keepdims=True))
        alpha = jnp.exp(m_run - m_new)
        p = jnp.exp(s - m_new)
        pb = p.astype(jnp.bfloat16)
        ones_col = jnp.ones((TTA, 1), jnp.bfloat16)
        l_ref[hh] = l_ref[hh] * alpha + jnp.dot(
            pb, ones_col, preferred_element_type=jnp.float32)
        pv = jnp.dot(pb, v_ref[hh], preferred_element_type=jnp.float32)
        acc_ref[hh] = acc_ref[hh] * alpha + pv
        m_ref[hh] = m_new

    @pl.when(diag)
    def _():
        for hh in range(HB):
            o_ref[hh] = acc_ref[hh] / l_ref[hh]


def _attn(q3, k3, v3):
    imap = []
    jmap = []
    for i in range(NBQ):
        for j in range(i + 1):
            imap.append(i)
            jmap.append(j)
    imap = jnp.asarray(imap, jnp.int32)
    jmap = jnp.asarray(jmap, jnp.int32)
    grid_spec = pltpu.PrefetchScalarGridSpec(
        num_scalar_prefetch=2,
        grid=(H // HB, NSTEP),
        in_specs=[
            pl.BlockSpec((HB, TTA, HD), lambda h, s, im, jm: (h, im[s], 0)),
            pl.BlockSpec((HB, TTA, HD), lambda h, s, im, jm: (h, jm[s], 0)),
            pl.BlockSpec((HB, TTA, HD), lambda h, s, im, jm: (h, jm[s], 0)),
        ],
        out_specs=pl.BlockSpec((HB, TTA, HD), lambda h, s, im, jm: (h, im[s], 0)),
        scratch_shapes=[
            pltpu.VMEM((HB, TTA, HD), jnp.float32),
            pltpu.VMEM((HB, TTA, 1), jnp.float32),
            pltpu.VMEM((HB, TTA, 1), jnp.float32),
        ],
    )
    return pl.pallas_call(
        _k2_body,
        grid_spec=grid_spec,
        out_shape=jax.ShapeDtypeStruct((H, T, HD), jnp.float32),
    )(imap, jmap, q3, k3, v3)


def _k3_body(ao_ref, x_ref, wp_ref, bp_ref, g_ref, b_ref, wr_ref, br_ref,
             y_ref, xn_ref, meta_ref, cnt_ref, carry):
    i = pl.program_id(0)

    @pl.when(i == 0)
    def _():
        carry[...] = jnp.zeros_like(carry)

    ao2 = jnp.concatenate([ao_ref[h] for h in range(H)], axis=1)
    y = jnp.dot(ao2.astype(jnp.bfloat16),
                wp_ref[...].astype(jnp.bfloat16),
                preferred_element_type=jnp.float32)
    y = y + bp_ref[...] + x_ref[...]
    y_ref[...] = y
    mu = jnp.mean(y, axis=1, keepdims=True)
    var = jnp.mean((y - mu) ** 2, axis=1, keepdims=True)
    xn = (y - mu) * lax.rsqrt(var + 1e-5) * g_ref[...] + b_ref[...]
    xn_ref[...] = xn

    logits = jnp.dot(xn, wr_ref[...], preferred_element_type=jnp.float32)
    logits = logits + br_ref[...]
    lane = lax.broadcasted_iota(jnp.int32, (TT, 128), 1)
    v0 = jnp.max(logits, axis=1, keepdims=True)
    e0 = jnp.min(jnp.where(logits == v0, lane, 128), axis=1, keepdims=True)
    l2 = jnp.where(lane == e0, NEG, logits)
    v1 = jnp.max(l2, axis=1, keepdims=True)
    e1 = jnp.min(jnp.where(l2 == v1, lane, 128), axis=1, keepdims=True)
    bexp = jnp.exp(v1 - v0)
    w0 = 1.0 / (1.0 + bexp)
    w1 = bexp * w0

    m0 = (lane == e0).astype(jnp.float32)
    m1 = (lane == e1).astype(jnp.float32)
    m = m0 + m1
    r = lax.broadcasted_iota(jnp.int32, (TT, TT), 0)
    c = lax.broadcasted_iota(jnp.int32, (TT, TT), 1)
    trilS = (r > c).astype(jnp.float32)
    rank = jnp.dot(trilS, m, preferred_element_type=jnp.float32) + carry[...]
    rank0 = jnp.sum(m0 * rank, axis=1, keepdims=True)
    rank1 = jnp.sum(m1 * rank, axis=1, keepdims=True)
    newc = carry[...] + jnp.sum(m, axis=0, keepdims=True)
    carry[...] = newc
    cnt_ref[...] = newc

    e0f = e0.astype(jnp.float32)
    e1f = e1.astype(jnp.float32)
    meta = (jnp.where(lane == 0, e0f, 0.0) + jnp.where(lane == 1, e1f, 0.0)
            + jnp.where(lane == 2, w0, 0.0) + jnp.where(lane == 3, w1, 0.0)
            + jnp.where(lane == 4, rank0, 0.0) + jnp.where(lane == 5, rank1, 0.0))
    meta_ref[...] = meta


def _proj_router(ao2, x2, wproj, bproj, ln2_g, ln2_b, wr_pad, br_pad):
    return pl.pallas_call(
        _k3_body,
        grid=(T // TT,),
        in_specs=[
            pl.BlockSpec((H, TT, HD), lambda i: (0, i, 0)),
            pl.BlockSpec((TT, D), lambda i: (i, 0)),
            pl.BlockSpec((D, D), lambda i: (0, 0)),
            pl.BlockSpec((1, D), lambda i: (0, 0)),
            pl.BlockSpec((1, D), lambda i: (0, 0)),
            pl.BlockSpec((1, D), lambda i: (0, 0)),
            pl.BlockSpec((D, 128), lambda i: (0, 0)),
            pl.BlockSpec((1, 128), lambda i: (0, 0)),
        ],
        out_specs=[
            pl.BlockSpec((TT, D), lambda i: (i, 0)),
            pl.BlockSpec((TT, D), lambda i: (i, 0)),
            pl.BlockSpec((TT, 128), lambda i: (i, 0)),
            pl.BlockSpec((1, 128), lambda i: (0, 0)),
        ],
        out_shape=[
            jax.ShapeDtypeStruct((T, D), jnp.float32),
            jax.ShapeDtypeStruct((T, D), jnp.float32),
            jax.ShapeDtypeStruct((T, 128), jnp.float32),
            jax.ShapeDtypeStruct((1, 128), jnp.float32),
        ],
        scratch_shapes=[pltpu.VMEM((1, 128), jnp.float32)],
    )(ao2, x2, wproj, bproj.reshape(1, D), ln2_g.reshape(1, D),
      ln2_b.reshape(1, D), wr_pad, br_pad)


def _k4_body(cnt_ref, meta_ref, meta2_ref, small_ref):
    lane = lax.broadcasted_iota(jnp.int32, (1, 128), 1)
    cnt = cnt_ref[...]
    p = jnp.floor((cnt + (TILE - 1.0)) * (1.0 / TILE)) * TILE
    lt = (lax.broadcasted_iota(jnp.int32, (128, 128), 0)
          < lax.broadcasted_iota(jnp.int32, (128, 128), 1)).astype(jnp.float32)
    P = jnp.dot(p, lt, preferred_element_type=jnp.float32)
    C = P + p
    total = jnp.sum(p, axis=1, keepdims=True)

    lane2 = lax.broadcasted_iota(jnp.int32, (TT, 128), 1)
    meta = meta_ref[...]
    e0f = jnp.sum(jnp.where(lane2 == 0, meta, 0.0), axis=1, keepdims=True)
    e1f = jnp.sum(jnp.where(lane2 == 1, meta, 0.0), axis=1, keepdims=True)
    rank0 = jnp.sum(jnp.where(lane2 == 4, meta, 0.0), axis=1, keepdims=True)
    rank1 = jnp.sum(jnp.where(lane2 == 5, meta, 0.0), axis=1, keepdims=True)
    lane2f = lane2.astype(jnp.float32)
    P0 = jnp.sum(jnp.where(lane2f == e0f, P, 0.0), axis=1, keepdims=True)
    P1 = jnp.sum(jnp.where(lane2f == e1f, P, 0.0), axis=1, keepdims=True)
    d0 = P0 + rank0
    d1 = P1 + rank1
    meta2_ref[...] = (jnp.where(lane2 == 0, d0, 0.0)
                      + jnp.where(lane2 == 1, d1, 0.0))

    thresh = lane.astype(jnp.float32) * TILE
    te = jnp.zeros((1, 128), jnp.float32)
    for e in range(E):
        Ce = jnp.sum(jnp.where(lane == e, C, 0.0), axis=1, keepdims=True)
        te = te + (Ce <= thresh).astype(jnp.float32)
    te = jnp.minimum(te, float(E - 1))
    small_ref[...] = (jnp.where(lane < NTILES, te, 0.0)
                      + jnp.where(lane == 30, total, 0.0))


def _routing_meta(cnt, meta):
    return pl.pallas_call(
        _k4_body,
        grid=(T // TT,),
        in_specs=[
            pl.BlockSpec((1, 128), lambda i: (0, 0)),
            pl.BlockSpec((TT, 128), lambda i: (i, 0)),
        ],
        out_specs=[
            pl.BlockSpec((TT, 128), lambda i: (i, 0)),
            pl.BlockSpec((1, 128), lambda i: (0, 0)),
        ],
        out_shape=[
            jax.ShapeDtypeStruct((T, 128), jnp.float32),
            jax.ShapeDtypeStruct((1, 128), jnp.float32),
        ],
    )(cnt, meta)


def _k6_body(te_ref, act_ref, gx_ref, w1_ref, w2_ref, w3_ref, b1_ref, b2_ref,
             b3_ref, wb_ref, o_ref):
    i = pl.program_id(0)

    @pl.when(act_ref[i] == 1)
    def _():
        gx = gx_ref[...].astype(jnp.bfloat16)
        w1b = w1_ref[0].astype(jnp.bfloat16)
        w2b = w2_ref[0].astype(jnp.bfloat16)
        h1 = jnp.dot(gx, w1b, preferred_element_type=jnp.float32) + b1_ref[0]
        h2 = jnp.dot(gx, w2b, preferred_element_type=jnp.float32) + b2_ref[0]
        sw = (h1 * (h2 * jax.nn.sigmoid(h2))).astype(jnp.bfloat16)
        w3b = w3_ref[0].astype(jnp.bfloat16)
        o = jnp.dot(sw, w3b, preferred_element_type=jnp.float32) + b3_ref[0]
        o_ref[...] = o * wb_ref[:, 0:1]


def _experts(te, act, gx, W1, b1, W2, b2, W3, b3, wb16):
    grid_spec = pltpu.PrefetchScalarGridSpec(
        num_scalar_prefetch=2,
        grid=(NTILES,),
        in_specs=[
            pl.BlockSpec((TILE, D), lambda i, te, act: (i, 0)),
            pl.BlockSpec((1, D, HID), lambda i, te, act: (te[i], 0, 0)),
            pl.BlockSpec((1, D, HID), lambda i, te, act: (te[i], 0, 0)),
            pl.BlockSpec((1, HID, D), lambda i, te, act: (te[i], 0, 0)),
            pl.BlockSpec((1, 1, HID), lambda i, te, act: (te[i], 0, 0)),
            pl.BlockSpec((1, 1, HID), lambda i, te, act: (te[i], 0, 0)),
            pl.BlockSpec((1, 1, D), lambda i, te, act: (te[i], 0, 0)),
            pl.BlockSpec((TILE, 128), lambda i, te, act: (i, 0)),
        ],
        out_specs=pl.BlockSpec((TILE, D), lambda i, te, act: (i, 0)),
    )
    return pl.pallas_call(
        _k6_body,
        grid_spec=grid_spec,
        out_shape=jax.ShapeDtypeStruct((NSLOT, D), jnp.float32),
    )(te, act, gx, W1, W2, W3, b1.reshape(E, 1, HID), b2.reshape(E, 1, HID),
      b3.reshape(E, 1, D), wb16)


NW = 32
ROWS_W = T // NW
CH = 32


def _k5_body(xn2_hbm, d0_hbm, d1_hbm, w016_hbm, w116_hbm, gx_hbm, wb16_hbm,
             rows_v, i0, i1, w0r, w1r, sem0, sem1, sem2, sem3):
    wid = lax.axis_index("s") * 2 + lax.axis_index("c")
    base = wid * ROWS_W
    pltpu.sync_copy(d0_hbm.at[pl.ds(base, ROWS_W)], i0)
    pltpu.sync_copy(d1_hbm.at[pl.ds(base, ROWS_W)], i1)
    pltpu.sync_copy(xn2_hbm.at[pl.ds(base, ROWS_W), :], rows_v)
    pltpu.sync_copy(w016_hbm.at[pl.ds(base, ROWS_W), :], w0r)
    pltpu.sync_copy(w116_hbm.at[pl.ds(base, ROWS_W), :], w1r)
    a = pltpu.async_copy(rows_v, gx_hbm.at[i0], sem0)
    b = pltpu.async_copy(rows_v, gx_hbm.at[i1], sem1)
    c = pltpu.async_copy(w0r, wb16_hbm.at[i0], sem2)
    d = pltpu.async_copy(w1r, wb16_hbm.at[i1], sem3)
    a.wait()
    b.wait()
    c.wait()
    d.wait()


def _dispatch(xn2, d0, d1, w016, w116):
    mesh = plsc.VectorSubcoreMesh(core_axis_name="c", subcore_axis_name="s", num_cores=2, num_subcores=16)
    fn = pl.kernel(
        _k5_body,
        out_type=(
            jax.ShapeDtypeStruct((NSLOT, D), jnp.float32),
            jax.ShapeDtypeStruct((NSLOT, 128), jnp.float32),
        ),
        mesh=mesh,
        scratch_types=[
            pltpu.VMEM((ROWS_W, D), jnp.float32),
            pltpu.VMEM((ROWS_W,), jnp.int32),
            pltpu.VMEM((ROWS_W,), jnp.int32),
            pltpu.VMEM((ROWS_W, 128), jnp.float32),
            pltpu.VMEM((ROWS_W, 128), jnp.float32),
            pltpu.SemaphoreType.DMA,
            pltpu.SemaphoreType.DMA,
            pltpu.SemaphoreType.DMA,
            pltpu.SemaphoreType.DMA,
        ],
    )
    return fn(xn2, d0, d1, w016, w116)


def _k7_body(y_hbm, obuf_hbm, d0_hbm, d1_hbm, out_hbm,
             r0, r1, ry, i0, i1, sem0, sem1):
    wid = lax.axis_index("s") * 2 + lax.axis_index("c")
    for c in range(ROWS_W // CH):
        b = wid * ROWS_W + c * CH
        pltpu.sync_copy(d0_hbm.at[pl.ds(b, CH)], i0)
        pltpu.sync_copy(d1_hbm.at[pl.ds(b, CH)], i1)
        a1 = pltpu.async_copy(obuf_hbm.at[i0], r0, sem0)
        a2 = pltpu.async_copy(obuf_hbm.at[i1], r1, sem1)
        pltpu.sync_copy(y_hbm.at[pl.ds(b, CH), :], ry)
        a1.wait()
        a2.wait()

        def jb(j, _):
            def kb(k, _):
                s = pl.ds(k * 16, 16)
                ry[j, s] = ry[j, s] + r0[j, s] + r1[j, s]
                return 0

            return lax.fori_loop(0, D // 16, kb, 0)

        lax.fori_loop(0, CH, jb, 0)
        pltpu.sync_copy(ry, out_hbm.at[pl.ds(b, CH), :])


def _combine(y, obuf, d0, d1):
    mesh = plsc.VectorSubcoreMesh(core_axis_name="c", subcore_axis_name="s", num_cores=2, num_subcores=16)
    fn = pl.kernel(
        _k7_body,
        out_type=jax.ShapeDtypeStruct((T, D), jnp.float32),
        mesh=mesh,
        scratch_types=[
            pltpu.VMEM((CH, D), jnp.float32),
            pltpu.VMEM((CH, D), jnp.float32),
            pltpu.VMEM((CH, D), jnp.float32),
            pltpu.VMEM((CH,), jnp.int32),
            pltpu.VMEM((CH,), jnp.int32),
            pltpu.SemaphoreType.DMA,
            pltpu.SemaphoreType.DMA,
        ],
    )
    return fn(y, obuf, d0, d1)


def kernel(x, ln1_g, ln1_b, Wq, Wk, Wv, Wproj, bproj, ln2_g, ln2_b,
           Wr, br, W1, b1, W2, b2, W3, b3):
    x2 = x.reshape(T, D)

    wqkv = jnp.concatenate([
        Wq.astype(jnp.bfloat16).transpose(1, 0, 2).reshape(D, D),
        Wk.astype(jnp.bfloat16).transpose(1, 0, 2).reshape(D, D),
        Wv.astype(jnp.bfloat16).transpose(1, 0, 2).reshape(D, D),
    ], axis=1)
    q3, k3, v3 = _qkv(x2, ln1_g, ln1_b, wqkv)

    ao = _attn(q3, k3, v3)

    wr_pad = jnp.zeros((D, 128), jnp.float32).at[:, :E].set(Wr)
    br_pad = jnp.full((1, 128), NEG, jnp.float32).at[0, :E].set(br)
    y, xn2, meta, cnt = _proj_router(ao, x2, Wproj, bproj, ln2_g, ln2_b,
                                     wr_pad, br_pad)
    meta2, small = _routing_meta(cnt, meta)

    d0 = meta2[:, 0].astype(jnp.int32)
    d1 = meta2[:, 1].astype(jnp.int32)
    w0 = meta[:, 2]
    w1 = meta[:, 3]
    te = small[0, :NTILES].astype(jnp.int32)
    total = small[0, 30]
    act = (jnp.arange(NTILES, dtype=jnp.float32) * TILE < total).astype(jnp.int32)

    w016 = jnp.broadcast_to(w0[:, None], (T, 128))
    w116 = jnp.broadcast_to(w1[:, None], (T, 128))
    gx, wb16 = _dispatch(xn2, d0, d1, w016, w116)

    obuf = _experts(te, act, gx, W1, b1, W2, b2, W3, b3, wb16)

    out = _combine(y, obuf, d0, d1)
    return out.reshape(B, T, D)


# speedup vs baseline: 1.0021x; 1.0021x over previous
import functools

import jax
import jax.numpy as jnp
from jax import lax
from jax.experimental import pallas as pl
from jax.experimental.pallas import tpu as pltpu
from jax.experimental.pallas import tpu_sc as plsc

B, T, D = 1, 2048, 768
H = 12
HD = D // H
E = 8
HID = int(4 * D * 2 / 3)

TT = 256
TILE = 256
NTILES = 23
NSLOT = NTILES * TILE
NEG = -1e30


def _k1_body(x_ref, g_ref, b_ref, w_ref, q_ref, k_ref, v_ref):
    x = x_ref[...]
    mu = jnp.mean(x, axis=1, keepdims=True)
    var = jnp.mean((x - mu) ** 2, axis=1, keepdims=True)
    xn = (x - mu) * lax.rsqrt(var + 1e-5) * g_ref[...] + b_ref[...]
    qkv = jnp.dot(xn.astype(jnp.bfloat16), w_ref[...],
                  preferred_element_type=jnp.float32).astype(jnp.bfloat16)
    for h in range(H):
        q_ref[h] = qkv[:, h * HD:(h + 1) * HD]
        k_ref[h] = qkv[:, D + h * HD:D + (h + 1) * HD]
        v_ref[h] = qkv[:, 2 * D + h * HD:2 * D + (h + 1) * HD]


def _qkv(x2, ln1_g, ln1_b, wqkv):
    return pl.pallas_call(
        _k1_body,
        grid=(T // TT,),
        in_specs=[
            pl.BlockSpec((TT, D), lambda i: (i, 0)),
            pl.BlockSpec((1, D), lambda i: (0, 0)),
            pl.BlockSpec((1, D), lambda i: (0, 0)),
            pl.BlockSpec((D, 3 * D), lambda i: (0, 0)),
        ],
        out_specs=[
            pl.BlockSpec((H, TT, HD), lambda i: (0, i, 0)),
            pl.BlockSpec((H, TT, HD), lambda i: (0, i, 0)),
            pl.BlockSpec((H, TT, HD), lambda i: (0, i, 0)),
        ],
        out_shape=[
            jax.ShapeDtypeStruct((H, T, HD), jnp.bfloat16),
            jax.ShapeDtypeStruct((H, T, HD), jnp.bfloat16),
            jax.ShapeDtypeStruct((H, T, HD), jnp.bfloat16),
        ],
    )(x2, ln1_g.reshape(1, D), ln1_b.reshape(1, D), wqkv)


TTA = 1024
NBQ = T // TTA
NSTEP = NBQ * (NBQ + 1) // 2
HB = 3


def _k2_body(imap_ref, jmap_ref, q_ref, k_ref, v_ref, o_ref,
             acc_ref, m_ref, l_ref):
    s_idx = pl.program_id(1)
    i = imap_ref[s_idx]
    j = jmap_ref[s_idx]

    @pl.when(j == 0)
    def _():
        acc_ref[...] = jnp.zeros_like(acc_ref)
        m_ref[...] = jnp.full_like(m_ref, NEG)
        l_ref[...] = jnp.zeros_like(l_ref)

    rowL = lax.broadcasted_iota(jnp.int32, (TTA, TTA), 0)
    colL = lax.broadcasted_iota(jnp.int32, (TTA, TTA), 1)
    diag = j == i
    for hh in range(HB):
        q = q_ref[hh] * jnp.bfloat16(1.0 / (HD ** 0.5))
        k = k_ref[hh]
        s = lax.dot_general(q, k, (((1,), (1,)), ((), ())),
                            preferred_element_type=jnp.float32)
        s = jnp.where(jnp.logical_or(jnp.logical_not(diag), colL <= rowL),
                      s, NEG)
        m_run = m_ref[hh]
        m_new = jnp.maximum(m_run, jnp.max(s, axis=1, keepdims=True))
        alpha = jnp.exp(m_run - m_new)
        p = jnp.exp(s - m_new)
        pb = p.astype(jnp.bfloat16)
        ones_col = jnp.ones((TTA, 1), jnp.bfloat16)
        l_ref[hh] = l_ref[hh] * alpha + jnp.dot(
            pb, ones_col, preferred_element_type=jnp.float32)
        pv = jnp.dot(pb, v_ref[hh], preferred_element_type=jnp.float32)
        acc_ref[hh] = acc_ref[hh] * alpha + pv
        m_ref[hh] = m_new

    @pl.when(diag)
    def _():
        for hh in range(HB):
            o_ref[hh] = acc_ref[hh] / l_ref[hh]


def _attn(q3, k3, v3):
    imap = []
    jmap = []
    for i in range(NBQ):
        for j in range(i + 1):
            imap.append(i)
            jmap.append(j)
    imap = jnp.asarray(imap, jnp.int32)
    jmap = jnp.asarray(jmap, jnp.int32)
    grid_spec = pltpu.PrefetchScalarGridSpec(
        num_scalar_prefetch=2,
        grid=(H // HB, NSTEP),
        in_specs=[
            pl.BlockSpec((HB, TTA, HD), lambda h, s, im, jm: (h, im[s], 0)),
            pl.BlockSpec((HB, TTA, HD), lambda h, s, im, jm: (h, jm[s], 0)),
            pl.BlockSpec((HB, TTA, HD), lambda h, s, im, jm: (h, jm[s], 0)),
        ],
        out_specs=pl.BlockSpec((HB, TTA, HD), lambda h, s, im, jm: (h, im[s], 0)),
        scratch_shapes=[
            pltpu.VMEM((HB, TTA, HD), jnp.float32),
            pltpu.VMEM((HB, TTA, 1), jnp.float32),
            pltpu.VMEM((HB, TTA, 1), jnp.float32),
        ],
    )
    return pl.pallas_call(
        _k2_body,
        grid_spec=grid_spec,
        out_shape=jax.ShapeDtypeStruct((H, T, HD), jnp.float32),
    )(imap, jmap, q3, k3, v3)


def _k3_body(ao_ref, x_ref, wp_ref, bp_ref, g_ref, b_ref, wr_ref, br_ref,
             y_ref, xn_ref, meta_ref, cnt_ref, carry):
    i = pl.program_id(0)

    @pl.when(i == 0)
    def _():
        carry[...] = jnp.zeros_like(carry)

    ao2 = jnp.concatenate([ao_ref[h] for h in range(H)], axis=1)
    y = jnp.dot(ao2.astype(jnp.bfloat16),
                wp_ref[...].astype(jnp.bfloat16),
                preferred_element_type=jnp.float32)
    y = y + bp_ref[...] + x_ref[...]
    y_ref[...] = y
    mu = jnp.mean(y, axis=1, keepdims=True)
    var = jnp.mean((y - mu) ** 2, axis=1, keepdims=True)
    xn = (y - mu) * lax.rsqrt(var + 1e-5) * g_ref[...] + b_ref[...]
    xn_ref[...] = xn

    logits = jnp.dot(xn, wr_ref[...], preferred_element_type=jnp.float32)
    logits = logits + br_ref[...]
    lane = lax.broadcasted_iota(jnp.int32, (TT, 128), 1)
    v0 = jnp.max(logits, axis=1, keepdims=True)
    e0 = jnp.min(jnp.where(logits == v0, lane, 128), axis=1, keepdims=True)
    l2 = jnp.where(lane == e0, NEG, logits)
    v1 = jnp.max(l2, axis=1, keepdims=True)
    e1 = jnp.min(jnp.where(l2 == v1, lane, 128), axis=1, keepdims=True)
    bexp = jnp.exp(v1 - v0)
    w0 = 1.0 / (1.0 + bexp)
    w1 = bexp * w0

    m0 = (lane == e0).astype(jnp.float32)
    m1 = (lane == e1).astype(jnp.float32)
    m = m0 + m1
    r = lax.broadcasted_iota(jnp.int32, (TT, TT), 0)
    c = lax.broadcasted_iota(jnp.int32, (TT, TT), 1)
    trilS = (r > c).astype(jnp.float32)
    rank = jnp.dot(trilS, m, preferred_element_type=jnp.float32) + carry[...]
    rank0 = jnp.sum(m0 * rank, axis=1, keepdims=True)
    rank1 = jnp.sum(m1 * rank, axis=1, keepdims=True)
    newc = carry[...] + jnp.sum(m, axis=0, keepdims=True)
    carry[...] = newc
    cnt_ref[...] = newc

    e0f = e0.astype(jnp.float32)
    e1f = e1.astype(jnp.float32)
    meta = (jnp.where(lane == 0, e0f, 0.0) + jnp.where(lane == 1, e1f, 0.0)
            + jnp.where(lane == 2, w0, 0.0) + jnp.where(lane == 3, w1, 0.0)
            + jnp.where(lane == 4, rank0, 0.0) + jnp.where(lane == 5, rank1, 0.0))
    meta_ref[...] = meta


def _proj_router(ao2, x2, wproj, bproj, ln2_g, ln2_b, wr_pad, br_pad):
    return pl.pallas_call(
        _k3_body,
        grid=(T // TT,),
        in_specs=[
            pl.BlockSpec((H, TT, HD), lambda i: (0, i, 0)),
            pl.BlockSpec((TT, D), lambda i: (i, 0)),
            pl.BlockSpec((D, D), lambda i: (0, 0)),
            pl.BlockSpec((1, D), lambda i: (0, 0)),
            pl.BlockSpec((1, D), lambda i: (0, 0)),
            pl.BlockSpec((1, D), lambda i: (0, 0)),
            pl.BlockSpec((D, 128), lambda i: (0, 0)),
            pl.BlockSpec((1, 128), lambda i: (0, 0)),
        ],
        out_specs=[
            pl.BlockSpec((TT, D), lambda i: (i, 0)),
            pl.BlockSpec((TT, D), lambda i: (i, 0)),
            pl.BlockSpec((TT, 128), lambda i: (i, 0)),
            pl.BlockSpec((1, 128), lambda i: (0, 0)),
        ],
        out_shape=[
            jax.ShapeDtypeStruct((T, D), jnp.float32),
            jax.ShapeDtypeStruct((T, D), jnp.float32),
            jax.ShapeDtypeStruct((T, 128), jnp.float32),
            jax.ShapeDtypeStruct((1, 128), jnp.float32),
        ],
        scratch_shapes=[pltpu.VMEM((1, 128), jnp.float32)],
    )(ao2, x2, wproj, bproj.reshape(1, D), ln2_g.reshape(1, D),
      ln2_b.reshape(1, D), wr_pad, br_pad)


def _k4_body(cnt_ref, meta_ref, meta2_ref, small_ref):
    lane = lax.broadcasted_iota(jnp.int32, (1, 128), 1)
    cnt = cnt_ref[...]
    p = jnp.floor((cnt + (TILE - 1.0)) * (1.0 / TILE)) * TILE
    lt = (lax.broadcasted_iota(jnp.int32, (128, 128), 0)
          < lax.broadcasted_iota(jnp.int32, (128, 128), 1)).astype(jnp.float32)
    P = jnp.dot(p, lt, preferred_element_type=jnp.float32)
    C = P + p
    total = jnp.sum(p, axis=1, keepdims=True)

    lane2 = lax.broadcasted_iota(jnp.int32, (TT, 128), 1)
    meta = meta_ref[...]
    e0f = jnp.sum(jnp.where(lane2 == 0, meta, 0.0), axis=1, keepdims=True)
    e1f = jnp.sum(jnp.where(lane2 == 1, meta, 0.0), axis=1, keepdims=True)
    rank0 = jnp.sum(jnp.where(lane2 == 4, meta, 0.0), axis=1, keepdims=True)
    rank1 = jnp.sum(jnp.where(lane2 == 5, meta, 0.0), axis=1, keepdims=True)
    lane2f = lane2.astype(jnp.float32)
    P0 = jnp.sum(jnp.where(lane2f == e0f, P, 0.0), axis=1, keepdims=True)
    P1 = jnp.sum(jnp.where(lane2f == e1f, P, 0.0), axis=1, keepdims=True)
    d0 = P0 + rank0
    d1 = P1 + rank1
    meta2_ref[...] = (jnp.where(lane2 == 0, d0, 0.0)
                      + jnp.where(lane2 == 1, d1, 0.0))

    thresh = lane.astype(jnp.float32) * TILE
    te = jnp.zeros((1, 128), jnp.float32)
    for e in range(E):
        Ce = jnp.sum(jnp.where(lane == e, C, 0.0), axis=1, keepdims=True)
        te = te + (Ce <= thresh).astype(jnp.float32)
    te = jnp.minimum(te, float(E - 1))
    small_ref[...] = (jnp.where(lane < NTILES, te, 0.0)
                      + jnp.where(lane == 30, total, 0.0))


def _routing_meta(cnt, meta):
    return pl.pallas_call(
        _k4_body,
        grid=(T // TT,),
        in_specs=[
            pl.BlockSpec((1, 128), lambda i: (0, 0)),
            pl.BlockSpec((TT, 128), lambda i: (i, 0)),
        ],
        out_specs=[
            pl.BlockSpec((TT, 128), lambda i: (i, 0)),
            pl.BlockSpec((1, 128), lambda i: (0, 0)),
        ],
        out_shape=[
            jax.ShapeDtypeStruct((T, 128), jnp.float32),
            jax.ShapeDtypeStruct((1, 128), jnp.float32),
        ],
    )(cnt, meta)


def _k6_body(te_ref, act_ref, gx_ref, w1_ref, w2_ref, w3_ref, b1_ref, b2_ref,
             b3_ref, wb_ref, o_ref):
    i = pl.program_id(0)

    @pl.when(act_ref[i] == 1)
    def _():
        gx = gx_ref[...].astype(jnp.bfloat16)
        w1b = w1_ref[0].astype(jnp.bfloat16)
        w2b = w2_ref[0].astype(jnp.bfloat16)
        h1 = jnp.dot(gx, w1b, preferred_element_type=jnp.float32) + b1_ref[0]
        h2 = jnp.dot(gx, w2b, preferred_element_type=jnp.float32) + b2_ref[0]
        sw = (h1 * (h2 * jax.nn.sigmoid(h2))).astype(jnp.bfloat16)
        w3b = w3_ref[0].astype(jnp.bfloat16)
        o = jnp.dot(sw, w3b, preferred_element_type=jnp.float32) + b3_ref[0]
        o_ref[...] = o * wb_ref[:, 0:1]


def _experts(te, act, gx, W1, b1, W2, b2, W3, b3, wb16):
    grid_spec = pltpu.PrefetchScalarGridSpec(
        num_scalar_prefetch=2,
        grid=(NTILES,),
        in_specs=[
            pl.BlockSpec((TILE, D), lambda i, te, act: (i, 0)),
            pl.BlockSpec((1, D, HID), lambda i, te, act: (te[i], 0, 0)),
            pl.BlockSpec((1, D, HID), lambda i, te, act: (te[i], 0, 0)),
            pl.BlockSpec((1, HID, D), lambda i, te, act: (te[i], 0, 0)),
            pl.BlockSpec((1, 1, HID), lambda i, te, act: (te[i], 0, 0)),
            pl.BlockSpec((1, 1, HID), lambda i, te, act: (te[i], 0, 0)),
            pl.BlockSpec((1, 1, D), lambda i, te, act: (te[i], 0, 0)),
            pl.BlockSpec((TILE, 128), lambda i, te, act: (i, 0)),
        ],
        out_specs=pl.BlockSpec((TILE, D), lambda i, te, act: (i, 0)),
    )
    return pl.pallas_call(
        _k6_body,
        grid_spec=grid_spec,
        out_shape=jax.ShapeDtypeStruct((NSLOT, D), jnp.float32),
    )(te, act, gx, W1, W2, W3, b1.reshape(E, 1, HID), b2.reshape(E, 1, HID),
      b3.reshape(E, 1, D), wb16)


NW = 32
ROWS_W = T // NW
CH = 32


def _k5_body(xn2_hbm, d0_hbm, d1_hbm, w016_hbm, w116_hbm, gx_hbm, wb16_hbm,
             rows_v, i0, i1, w0r, w1r, sem0, sem1, sem2, sem3):
    wid = lax.axis_index("s") * 2 + lax.axis_index("c")
    base = wid * ROWS_W
    pltpu.sync_copy(d0_hbm.at[pl.ds(base, ROWS_W)], i0)
    pltpu.sync_copy(d1_hbm.at[pl.ds(base, ROWS_W)], i1)
    pltpu.sync_copy(xn2_hbm.at[pl.ds(base, ROWS_W), :], rows_v)
    pltpu.sync_copy(w016_hbm.at[pl.ds(base, ROWS_W), :], w0r)
    pltpu.sync_copy(w116_hbm.at[pl.ds(base, ROWS_W), :], w1r)
    a = pltpu.async_copy(rows_v, gx_hbm.at[i0], sem0)
    b = pltpu.async_copy(rows_v, gx_hbm.at[i1], sem1)
    c = pltpu.async_copy(w0r, wb16_hbm.at[i0], sem2)
    d = pltpu.async_copy(w1r, wb16_hbm.at[i1], sem3)
    a.wait()
    b.wait()
    c.wait()
    d.wait()


def _dispatch(xn2, d0, d1, w016, w116):
    mesh = plsc.VectorSubcoreMesh(core_axis_name="c", subcore_axis_name="s", num_cores=2, num_subcores=16)
    fn = pl.kernel(
        _k5_body,
        out_type=(
            jax.ShapeDtypeStruct((NSLOT, D), jnp.float32),
            jax.ShapeDtypeStruct((NSLOT, 128), jnp.float32),
        ),
        mesh=mesh,
        scratch_types=[
            pltpu.VMEM((ROWS_W, D), jnp.float32),
            pltpu.VMEM((ROWS_W,), jnp.int32),
            pltpu.VMEM((ROWS_W,), jnp.int32),
            pltpu.VMEM((ROWS_W, 128), jnp.float32),
            pltpu.VMEM((ROWS_W, 128), jnp.float32),
            pltpu.SemaphoreType.DMA,
            pltpu.SemaphoreType.DMA,
            pltpu.SemaphoreType.DMA,
            pltpu.SemaphoreType.DMA,
        ],
    )
    return fn(xn2, d0, d1, w016, w116)


def _k7_body(y_hbm, obuf_hbm, d0_hbm, d1_hbm, out_hbm,
             r0, r1, ry, i0, i1, sem0, sem1):
    wid = lax.axis_index("s") * 2 + lax.axis_index("c")
    for c in range(ROWS_W // CH):
        b = wid * ROWS_W + c * CH
        pltpu.sync_copy(d0_hbm.at[pl.ds(b, CH)], i0)
        pltpu.sync_copy(d1_hbm.at[pl.ds(b, CH)], i1)
        a1 = pltpu.async_copy(obuf_hbm.at[i0], r0, sem0)
        a2 = pltpu.async_copy(obuf_hbm.at[i1], r1, sem1)
        pltpu.sync_copy(y_hbm.at[pl.ds(b, CH), :], ry)
        a1.wait()
        a2.wait()

        def jb(j, _):
            def kb(k, _):
                s = pl.ds(k * 16, 16)
                ry[j, s] = ry[j, s] + r0[j, s] + r1[j, s]
                return 0

            return lax.fori_loop(0, D // 16, kb, 0)

        lax.fori_loop(0, CH, jb, 0)
        pltpu.sync_copy(ry, out_hbm.at[pl.ds(b, CH), :])


def _combine(y, obuf, d0, d1):
    mesh = plsc.VectorSubcoreMesh(core_axis_name="c", subcore_axis_name="s", num_cores=2, num_subcores=16)
    fn = pl.kernel(
        _k7_body,
        out_type=jax.ShapeDtypeStruct((T, D), jnp.float32),
        mesh=mesh,
        scratch_types=[
            pltpu.VMEM((CH, D), jnp.float32),
            pltpu.VMEM((CH, D), jnp.float32),
            pltpu.VMEM((CH, D), jnp.float32),
            pltpu.VMEM((CH,), jnp.int32),
            pltpu.VMEM((CH,), jnp.int32),
            pltpu.SemaphoreType.DMA,
            pltpu.SemaphoreType.DMA,
        ],
    )
    return fn(y, obuf, d0, d1)


def kernel(x, ln1_g, ln1_b, Wq, Wk, Wv, Wproj, bproj, ln2_g, ln2_b,
           Wr, br, W1, b1, W2, b2, W3, b3):
    x2 = x.reshape(T, D)

    wqkv = jnp.concatenate([
        Wq.astype(jnp.bfloat16).transpose(1, 0, 2).reshape(D, D),
        Wk.astype(jnp.bfloat16).transpose(1, 0, 2).reshape(D, D),
        Wv.astype(jnp.bfloat16).transpose(1, 0, 2).reshape(D, D),
    ], axis=1)
    q3, k3, v3 = _qkv(x2, ln1_g, ln1_b, wqkv)

    ao = _attn(q3, k3, v3)

    wr_pad = jnp.zeros((D, 128), jnp.float32).at[:, :E].set(Wr)
    br_pad = jnp.full((1, 128), NEG, jnp.float32).at[0, :E].set(br)
    y, xn2, meta, cnt = _proj_router(ao, x2, Wproj, bproj, ln2_g, ln2_b,
                                     wr_pad, br_pad)
    meta2, small = _routing_meta(cnt, meta)

    d0 = meta2[:, 0].astype(jnp.int32)
    d1 = meta2[:, 1].astype(jnp.int32)
    w0 = meta[:, 2]
    w1 = meta[:, 3]
    te = small[0, :NTILES].astype(jnp.int32)
    total = small[0, 30]
    act = (jnp.arange(NTILES, dtype=jnp.float32) * TILE < total).astype(jnp.int32)

    w016 = jnp.broadcast_to(w0[:, None], (T, 128))
    w116 = jnp.broadcast_to(w1[:, None], (T, 128))
    gx, wb16 = _dispatch(xn2, d0, d1, w016, w116)

    obuf = _experts(te, act, gx, W1, b1, W2, b2, W3, b3, wb16)

    out = _combine(y, obuf, d0, d1)
    return out.reshape(B, T, D)


# speedup vs baseline: 1.0264x; 1.0243x over previous
import functools

import jax
import jax.numpy as jnp
from jax import lax
from jax.experimental import pallas as pl
from jax.experimental.pallas import tpu as pltpu
from jax.experimental.pallas import tpu_sc as plsc

B, T, D = 1, 2048, 768
H = 12
HD = D // H
E = 8
HID = int(4 * D * 2 / 3)

TT = 256
TILE = 256
NTILES = 23
NSLOT = NTILES * TILE
NEG = -1e30


def _k1_body(x_ref, g_ref, b_ref, w_ref, q_ref, k_ref, v_ref):
    x = x_ref[...]
    mu = jnp.mean(x, axis=1, keepdims=True)
    var = jnp.mean((x - mu) ** 2, axis=1, keepdims=True)
    xn = (x - mu) * lax.rsqrt(var + 1e-5) * g_ref[...] + b_ref[...]
    qkv = jnp.dot(xn.astype(jnp.bfloat16), w_ref[...],
                  preferred_element_type=jnp.float32).astype(jnp.bfloat16)
    for h in range(H):
        q_ref[h] = qkv[:, h * HD:(h + 1) * HD]
        k_ref[h] = qkv[:, D + h * HD:D + (h + 1) * HD]
        v_ref[h] = qkv[:, 2 * D + h * HD:2 * D + (h + 1) * HD]


def _qkv(x2, ln1_g, ln1_b, wqkv):
    return pl.pallas_call(
        _k1_body,
        grid=(T // TT,),
        in_specs=[
            pl.BlockSpec((TT, D), lambda i: (i, 0)),
            pl.BlockSpec((1, D), lambda i: (0, 0)),
            pl.BlockSpec((1, D), lambda i: (0, 0)),
            pl.BlockSpec((D, 3 * D), lambda i: (0, 0)),
        ],
        out_specs=[
            pl.BlockSpec((H, TT, HD), lambda i: (0, i, 0)),
            pl.BlockSpec((H, TT, HD), lambda i: (0, i, 0)),
            pl.BlockSpec((H, TT, HD), lambda i: (0, i, 0)),
        ],
        out_shape=[
            jax.ShapeDtypeStruct((H, T, HD), jnp.bfloat16),
            jax.ShapeDtypeStruct((H, T, HD), jnp.bfloat16),
            jax.ShapeDtypeStruct((H, T, HD), jnp.bfloat16),
        ],
    )(x2, ln1_g.reshape(1, D), ln1_b.reshape(1, D), wqkv)


TTA = 1024
NBQ = T // TTA
NSTEP = NBQ * (NBQ + 1) // 2
HB = 3


def _k2_body(imap_ref, jmap_ref, q_ref, k_ref, v_ref, o_ref,
             acc_ref, m_ref, l_ref):
    s_idx = pl.program_id(1)
    i = imap_ref[s_idx]
    j = jmap_ref[s_idx]

    @pl.when(j == 0)
    def _():
        acc_ref[...] = jnp.zeros_like(acc_ref)
        m_ref[...] = jnp.full_like(m_ref, NEG)
        l_ref[...] = jnp.zeros_like(l_ref)

    rowL = lax.broadcasted_iota(jnp.int32, (TTA, TTA), 0)
    colL = lax.broadcasted_iota(jnp.int32, (TTA, TTA), 1)
    diag = j == i
    for hh in range(HB):
        q = q_ref[hh] * jnp.bfloat16(1.0 / (HD ** 0.5))
        k = k_ref[hh]
        s = lax.dot_general(q, k, (((1,), (1,)), ((), ())),
                            preferred_element_type=jnp.float32)
        s = jnp.where(jnp.logical_or(jnp.logical_not(diag), colL <= rowL),
                      s, NEG)
        m_run = m_ref[hh]
        m_new = jnp.maximum(m_run, jnp.max(s, axis=1, keepdims=True))
        alpha = jnp.exp(m_run - m_new)
        p = jnp.exp(s - m_new)
        pb = p.astype(jnp.bfloat16)
        ones_col = jnp.ones((TTA, 1), jnp.bfloat16)
        l_ref[hh] = l_ref[hh] * alpha + jnp.dot(
            pb, ones_col, preferred_element_type=jnp.float32)
        pv = jnp.dot(pb, v_ref[hh], preferred_element_type=jnp.float32)
        acc_ref[hh] = acc_ref[hh] * alpha + pv
        m_ref[hh] = m_new

    @pl.when(diag)
    def _():
        for hh in range(HB):
            o_ref[hh] = acc_ref[hh] / l_ref[hh]


def _attn(q3, k3, v3):
    imap = []
    jmap = []
    for i in range(NBQ):
        for j in range(i + 1):
            imap.append(i)
            jmap.append(j)
    imap = jnp.asarray(imap, jnp.int32)
    jmap = jnp.asarray(jmap, jnp.int32)
    grid_spec = pltpu.PrefetchScalarGridSpec(
        num_scalar_prefetch=2,
        grid=(H // HB, NSTEP),
        in_specs=[
            pl.BlockSpec((HB, TTA, HD), lambda h, s, im, jm: (h, im[s], 0)),
            pl.BlockSpec((HB, TTA, HD), lambda h, s, im, jm: (h, jm[s], 0)),
            pl.BlockSpec((HB, TTA, HD), lambda h, s, im, jm: (h, jm[s], 0)),
        ],
        out_specs=pl.BlockSpec((HB, TTA, HD), lambda h, s, im, jm: (h, im[s], 0)),
        scratch_shapes=[
            pltpu.VMEM((HB, TTA, HD), jnp.float32),
            pltpu.VMEM((HB, TTA, 1), jnp.float32),
            pltpu.VMEM((HB, TTA, 1), jnp.float32),
        ],
    )
    return pl.pallas_call(
        _k2_body,
        grid_spec=grid_spec,
        out_shape=jax.ShapeDtypeStruct((H, T, HD), jnp.float32),
    )(imap, jmap, q3, k3, v3)


def _k3_body(ao_ref, x_ref, wp_ref, bp_ref, g_ref, b_ref, wr_ref, br_ref,
             y_ref, xn_ref, meta_ref, cnt_ref, carry):
    i = pl.program_id(0)

    @pl.when(i == 0)
    def _():
        carry[...] = jnp.zeros_like(carry)

    ao2 = jnp.concatenate([ao_ref[h] for h in range(H)], axis=1)
    y = jnp.dot(ao2.astype(jnp.bfloat16),
                wp_ref[...].astype(jnp.bfloat16),
                preferred_element_type=jnp.float32)
    y = y + bp_ref[...] + x_ref[...]
    y_ref[...] = y
    mu = jnp.mean(y, axis=1, keepdims=True)
    var = jnp.mean((y - mu) ** 2, axis=1, keepdims=True)
    xn = (y - mu) * lax.rsqrt(var + 1e-5) * g_ref[...] + b_ref[...]
    xn_ref[...] = xn

    logits = jnp.dot(xn, wr_ref[...], preferred_element_type=jnp.float32)
    logits = logits + br_ref[...]
    lane = lax.broadcasted_iota(jnp.int32, (TT, 128), 1)
    v0 = jnp.max(logits, axis=1, keepdims=True)
    e0 = jnp.min(jnp.where(logits == v0, lane, 128), axis=1, keepdims=True)
    l2 = jnp.where(lane == e0, NEG, logits)
    v1 = jnp.max(l2, axis=1, keepdims=True)
    e1 = jnp.min(jnp.where(l2 == v1, lane, 128), axis=1, keepdims=True)
    bexp = jnp.exp(v1 - v0)
    w0 = 1.0 / (1.0 + bexp)
    w1 = bexp * w0

    m0 = (lane == e0).astype(jnp.float32)
    m1 = (lane == e1).astype(jnp.float32)
    m = m0 + m1
    r = lax.broadcasted_iota(jnp.int32, (TT, TT), 0)
    c = lax.broadcasted_iota(jnp.int32, (TT, TT), 1)
    trilS = (r > c).astype(jnp.float32)
    rank = jnp.dot(trilS, m, preferred_element_type=jnp.float32) + carry[...]
    rank0 = jnp.sum(m0 * rank, axis=1, keepdims=True)
    rank1 = jnp.sum(m1 * rank, axis=1, keepdims=True)
    newc = carry[...] + jnp.sum(m, axis=0, keepdims=True)
    carry[...] = newc
    cnt_ref[...] = newc

    e0f = e0.astype(jnp.float32)
    e1f = e1.astype(jnp.float32)
    meta = (jnp.where(lane == 0, e0f, 0.0) + jnp.where(lane == 1, e1f, 0.0)
            + jnp.where(lane == 2, w0, 0.0) + jnp.where(lane == 3, w1, 0.0)
            + jnp.where(lane == 4, rank0, 0.0) + jnp.where(lane == 5, rank1, 0.0))
    meta_ref[...] = meta


def _proj_router(ao2, x2, wproj, bproj, ln2_g, ln2_b, wr_pad, br_pad):
    return pl.pallas_call(
        _k3_body,
        grid=(T // TT,),
        in_specs=[
            pl.BlockSpec((H, TT, HD), lambda i: (0, i, 0)),
            pl.BlockSpec((TT, D), lambda i: (i, 0)),
            pl.BlockSpec((D, D), lambda i: (0, 0)),
            pl.BlockSpec((1, D), lambda i: (0, 0)),
            pl.BlockSpec((1, D), lambda i: (0, 0)),
            pl.BlockSpec((1, D), lambda i: (0, 0)),
            pl.BlockSpec((D, 128), lambda i: (0, 0)),
            pl.BlockSpec((1, 128), lambda i: (0, 0)),
        ],
        out_specs=[
            pl.BlockSpec((TT, D), lambda i: (i, 0)),
            pl.BlockSpec((TT, D), lambda i: (i, 0)),
            pl.BlockSpec((TT, 128), lambda i: (i, 0)),
            pl.BlockSpec((1, 128), lambda i: (0, 0)),
        ],
        out_shape=[
            jax.ShapeDtypeStruct((T, D), jnp.float32),
            jax.ShapeDtypeStruct((T, D), jnp.float32),
            jax.ShapeDtypeStruct((T, 128), jnp.float32),
            jax.ShapeDtypeStruct((1, 128), jnp.float32),
        ],
        scratch_shapes=[pltpu.VMEM((1, 128), jnp.float32)],
    )(ao2, x2, wproj, bproj.reshape(1, D), ln2_g.reshape(1, D),
      ln2_b.reshape(1, D), wr_pad, br_pad)


def _k4_body(cnt_ref, meta_ref, meta2_ref, small_ref):
    lane = lax.broadcasted_iota(jnp.int32, (1, 128), 1)
    cnt = cnt_ref[...]
    p = jnp.floor((cnt + (TILE - 1.0)) * (1.0 / TILE)) * TILE
    lt = (lax.broadcasted_iota(jnp.int32, (128, 128), 0)
          < lax.broadcasted_iota(jnp.int32, (128, 128), 1)).astype(jnp.float32)
    P = jnp.dot(p, lt, preferred_element_type=jnp.float32)
    C = P + p
    total = jnp.sum(p, axis=1, keepdims=True)

    lane2 = lax.broadcasted_iota(jnp.int32, (TT, 128), 1)
    meta = meta_ref[...]
    e0f = jnp.sum(jnp.where(lane2 == 0, meta, 0.0), axis=1, keepdims=True)
    e1f = jnp.sum(jnp.where(lane2 == 1, meta, 0.0), axis=1, keepdims=True)
    rank0 = jnp.sum(jnp.where(lane2 == 4, meta, 0.0), axis=1, keepdims=True)
    rank1 = jnp.sum(jnp.where(lane2 == 5, meta, 0.0), axis=1, keepdims=True)
    lane2f = lane2.astype(jnp.float32)
    P0 = jnp.sum(jnp.where(lane2f == e0f, P, 0.0), axis=1, keepdims=True)
    P1 = jnp.sum(jnp.where(lane2f == e1f, P, 0.0), axis=1, keepdims=True)
    d0 = P0 + rank0
    d1 = P1 + rank1
    meta2_ref[...] = (jnp.where(lane2 == 0, d0, 0.0)
                      + jnp.where(lane2 == 1, d1, 0.0))

    thresh = lane.astype(jnp.float32) * TILE
    te = jnp.zeros((1, 128), jnp.float32)
    for e in range(E):
        Ce = jnp.sum(jnp.where(lane == e, C, 0.0), axis=1, keepdims=True)
        te = te + (Ce <= thresh).astype(jnp.float32)
    te = jnp.minimum(te, float(E - 1))
    small_ref[...] = (jnp.where(lane < NTILES, te, 0.0)
                      + jnp.where(lane == 30, total, 0.0))


def _routing_meta(cnt, meta):
    return pl.pallas_call(
        _k4_body,
        grid=(T // TT,),
        in_specs=[
            pl.BlockSpec((1, 128), lambda i: (0, 0)),
            pl.BlockSpec((TT, 128), lambda i: (i, 0)),
        ],
        out_specs=[
            pl.BlockSpec((TT, 128), lambda i: (i, 0)),
            pl.BlockSpec((1, 128), lambda i: (0, 0)),
        ],
        out_shape=[
            jax.ShapeDtypeStruct((T, 128), jnp.float32),
            jax.ShapeDtypeStruct((1, 128), jnp.float32),
        ],
    )(cnt, meta)


def _k6_body(te_ref, act_ref, gx_ref, w1_ref, w2_ref, w3_ref, b1_ref, b2_ref,
             b3_ref, wb_ref, o_ref):
    i = pl.program_id(0)

    @pl.when(act_ref[i] == 1)
    def _():
        gx = gx_ref[...].astype(jnp.bfloat16)
        w1b = w1_ref[0].astype(jnp.bfloat16)
        w2b = w2_ref[0].astype(jnp.bfloat16)
        h1 = jnp.dot(gx, w1b, preferred_element_type=jnp.float32) + b1_ref[0]
        h2 = jnp.dot(gx, w2b, preferred_element_type=jnp.float32) + b2_ref[0]
        sw = (h1 * (h2 * jax.nn.sigmoid(h2))).astype(jnp.bfloat16)
        w3b = w3_ref[0].astype(jnp.bfloat16)
        o = jnp.dot(sw, w3b, preferred_element_type=jnp.float32) + b3_ref[0]
        o_ref[...] = o * wb_ref[:, 0:1]


def _experts(te, act, gx, W1, b1, W2, b2, W3, b3, wb16):
    grid_spec = pltpu.PrefetchScalarGridSpec(
        num_scalar_prefetch=2,
        grid=(NTILES,),
        in_specs=[
            pl.BlockSpec((TILE, D), lambda i, te, act: (i, 0)),
            pl.BlockSpec((1, D, HID), lambda i, te, act: (te[i], 0, 0)),
            pl.BlockSpec((1, D, HID), lambda i, te, act: (te[i], 0, 0)),
            pl.BlockSpec((1, HID, D), lambda i, te, act: (te[i], 0, 0)),
            pl.BlockSpec((1, 1, HID), lambda i, te, act: (te[i], 0, 0)),
            pl.BlockSpec((1, 1, HID), lambda i, te, act: (te[i], 0, 0)),
            pl.BlockSpec((1, 1, D), lambda i, te, act: (te[i], 0, 0)),
            pl.BlockSpec((TILE, 128), lambda i, te, act: (i, 0)),
        ],
        out_specs=pl.BlockSpec((TILE, D), lambda i, te, act: (i, 0)),
    )
    return pl.pallas_call(
        _k6_body,
        grid_spec=grid_spec,
        out_shape=jax.ShapeDtypeStruct((NSLOT, D), jnp.float32),
    )(te, act, gx, W1, W2, W3, b1.reshape(E, 1, HID), b2.reshape(E, 1, HID),
      b3.reshape(E, 1, D), wb16)


NW = 32
ROWS_W = T // NW
CH = 32


def _k5_body(xn2_hbm, d0_hbm, d1_hbm, w016_hbm, w116_hbm, gx_hbm, wb16_hbm,
             rows_v, i0, i1, w0r, w1r, sem0, sem1, sem2, sem3):
    wid = lax.axis_index("s") * 2 + lax.axis_index("c")
    base = wid * ROWS_W
    pltpu.sync_copy(d0_hbm.at[pl.ds(base, ROWS_W)], i0)
    pltpu.sync_copy(d1_hbm.at[pl.ds(base, ROWS_W)], i1)
    pltpu.sync_copy(xn2_hbm.at[pl.ds(base, ROWS_W), :], rows_v)
    pltpu.sync_copy(w016_hbm.at[pl.ds(base, ROWS_W), :], w0r)
    pltpu.sync_copy(w116_hbm.at[pl.ds(base, ROWS_W), :], w1r)
    a = pltpu.async_copy(rows_v, gx_hbm.at[i0], sem0)
    b = pltpu.async_copy(rows_v, gx_hbm.at[i1], sem1)
    c = pltpu.async_copy(w0r, wb16_hbm.at[i0], sem2)
    d = pltpu.async_copy(w1r, wb16_hbm.at[i1], sem3)
    a.wait()
    b.wait()
    c.wait()
    d.wait()


def _dispatch(xn2, d0, d1, w016, w116):
    mesh = plsc.VectorSubcoreMesh(core_axis_name="c", subcore_axis_name="s", num_cores=2, num_subcores=16)
    fn = pl.kernel(
        _k5_body,
        out_type=(
            jax.ShapeDtypeStruct((NSLOT, D), jnp.float32),
            jax.ShapeDtypeStruct((NSLOT, 128), jnp.float32),
        ),
        mesh=mesh,
        scratch_types=[
            pltpu.VMEM((ROWS_W, D), jnp.float32),
            pltpu.VMEM((ROWS_W,), jnp.int32),
            pltpu.VMEM((ROWS_W,), jnp.int32),
            pltpu.VMEM((ROWS_W, 128), jnp.float32),
            pltpu.VMEM((ROWS_W, 128), jnp.float32),
            pltpu.SemaphoreType.DMA,
            pltpu.SemaphoreType.DMA,
            pltpu.SemaphoreType.DMA,
            pltpu.SemaphoreType.DMA,
        ],
    )
    return fn(xn2, d0, d1, w016, w116)


def _k7_body(y_hbm, obuf_hbm, d0_hbm, d1_hbm, out_hbm,
             r0, r1, ry, i0, i1, sem0, sem1):
    wid = lax.axis_index("s") * 2 + lax.axis_index("c")
    for c in range(ROWS_W // CH):
        b = wid * ROWS_W + c * CH
        pltpu.sync_copy(d0_hbm.at[pl.ds(b, CH)], i0)
        pltpu.sync_copy(d1_hbm.at[pl.ds(b, CH)], i1)
        a1 = pltpu.async_copy(obuf_hbm.at[i0], r0, sem0)
        a2 = pltpu.async_copy(obuf_hbm.at[i1], r1, sem1)
        pltpu.sync_copy(y_hbm.at[pl.ds(b, CH), :], ry)
        a1.wait()
        a2.wait()

        def jb(j, _):
            for k in range(D // 16):
                s = pl.ds(k * 16, 16)
                ry[j, s] = ry[j, s] + r0[j, s] + r1[j, s]
            return 0

        lax.fori_loop(0, CH, jb, 0)
        pltpu.sync_copy(ry, out_hbm.at[pl.ds(b, CH), :])


def _combine(y, obuf, d0, d1):
    mesh = plsc.VectorSubcoreMesh(core_axis_name="c", subcore_axis_name="s", num_cores=2, num_subcores=16)
    fn = pl.kernel(
        _k7_body,
        out_type=jax.ShapeDtypeStruct((T, D), jnp.float32),
        mesh=mesh,
        scratch_types=[
            pltpu.VMEM((CH, D), jnp.float32),
            pltpu.VMEM((CH, D), jnp.float32),
            pltpu.VMEM((CH, D), jnp.float32),
            pltpu.VMEM((CH,), jnp.int32),
            pltpu.VMEM((CH,), jnp.int32),
            pltpu.SemaphoreType.DMA,
            pltpu.SemaphoreType.DMA,
        ],
    )
    return fn(y, obuf, d0, d1)


def kernel(x, ln1_g, ln1_b, Wq, Wk, Wv, Wproj, bproj, ln2_g, ln2_b,
           Wr, br, W1, b1, W2, b2, W3, b3):
    x2 = x.reshape(T, D)

    wqkv = jnp.concatenate([
        Wq.astype(jnp.bfloat16).transpose(1, 0, 2).reshape(D, D),
        Wk.astype(jnp.bfloat16).transpose(1, 0, 2).reshape(D, D),
        Wv.astype(jnp.bfloat16).transpose(1, 0, 2).reshape(D, D),
    ], axis=1)
    q3, k3, v3 = _qkv(x2, ln1_g, ln1_b, wqkv)

    ao = _attn(q3, k3, v3)

    wr_pad = jnp.zeros((D, 128), jnp.float32).at[:, :E].set(Wr)
    br_pad = jnp.full((1, 128), NEG, jnp.float32).at[0, :E].set(br)
    y, xn2, meta, cnt = _proj_router(ao, x2, Wproj, bproj, ln2_g, ln2_b,
                                     wr_pad, br_pad)
    meta2, small = _routing_meta(cnt, meta)

    d0 = meta2[:, 0].astype(jnp.int32)
    d1 = meta2[:, 1].astype(jnp.int32)
    w0 = meta[:, 2]
    w1 = meta[:, 3]
    te = small[0, :NTILES].astype(jnp.int32)
    total = small[0, 30]
    act = (jnp.arange(NTILES, dtype=jnp.float32) * TILE < total).astype(jnp.int32)

    w016 = jnp.broadcast_to(w0[:, None], (T, 128))
    w116 = jnp.broadcast_to(w1[:, None], (T, 128))
    gx, wb16 = _dispatch(xn2, d0, d1, w016, w116)

    obuf = _experts(te, act, gx, W1, b1, W2, b2, W3, b3, wb16)

    out = _combine(y, obuf, d0, d1)
    return out.reshape(B, T, D)
